# baseline (speedup 1.0000x reference)
import functools

import jax
import jax.numpy as jnp
from jax import lax
from jax.experimental import pallas as pl
from jax.experimental.pallas import tpu as pltpu
from jax.experimental.pallas import tpu_sc as plsc

_NUM_CODES = 8192
_DIM = 256
_ROWS = 8192
_TILE = 128
_GRID = _ROWS // _TILE
_HB = 8
_NC = 2
_NS = 16
_BPW = _ROWS // (_NC * _NS)
_CHUNK = 128

_LOG2E = 1.4426950408889634


def _dist_body(z_ref, e_ref, p_ref, idx_ref, loss_ref, acc_ref):
    z = z_ref[...]
    e = e_ref[...]
    z2 = jnp.sum(z * z, axis=1, keepdims=True)
    e2 = jnp.sum(e * e, axis=1)
    mm = lax.dot_general(z * -2.0, e, (((1,), (1,)), ((), ())),
                         preferred_element_type=jnp.float32)
    d = (z2 + e2[None, :]) + mm
    dmin = jnp.min(d, axis=1, keepdims=True)
    t = d - dmin
    ids8 = lax.broadcasted_iota(jnp.int32, (8, _NUM_CODES), 1).astype(
        jnp.float32)
    t3 = t.reshape(_TILE // 8, 8, _NUM_CODES)
    idxf = jnp.min(jnp.where(t3 == 0.0, ids8[None], jnp.float32(3e9)), axis=2)
    idx = idxf.reshape(_TILE).astype(jnp.int32)
    un = jnp.exp2(t * (-2.0 * _LOG2E))
    rcp = 1.0 / jnp.sum(un, axis=1, keepdims=True)
    p_ref[...] = un * rcp
    idx_ref[0, 0, :] = idx

    @pl.when(pl.program_id(0) == 0)
    def _():
        acc_ref[0] = 0.0

    acc_ref[0] += jnp.sum(dmin)

    @pl.when(pl.program_id(0) == _GRID - 1)
    def _():
        loss_ref[0, 0] = acc_ref[0] * (1.25 / (_ROWS * _DIM))


def _make_gather():
    mesh = plsc.VectorSubcoreMesh(core_axis_name="c", subcore_axis_name="s")

    @functools.partial(
        pl.kernel, mesh=mesh,
        out_type=jax.ShapeDtypeStruct((_ROWS, _DIM), jnp.float32),
        scratch_types=[
            pltpu.VMEM((_CHUNK,), jnp.int32),
            pltpu.VMEM((_CHUNK, _DIM), jnp.float32),
            pltpu.SemaphoreType.DMA,
        ],
    )
    def gather(emb_hbm, idx_hbm, out_hbm, idx_v, rows_v, sem):
        wid = lax.axis_index("s") * _NC + lax.axis_index("c")
        base = wid * _BPW
        for j in range(_BPW // _CHUNK):
            off = base + j * _CHUNK
            pltpu.sync_copy(
                idx_hbm.at[off // _TILE, 0, pl.ds(off % _TILE, _CHUNK)],
                idx_v)
            pltpu.async_copy(emb_hbm.at[idx_v], rows_v, sem).wait()
            pltpu.sync_copy(rows_v, out_hbm.at[pl.ds(off, _CHUNK)])

    return gather


@functools.cache
def _gather_fn():
    return _make_gather()


def kernel(z, embedding):
    b, c, h, w = z.shape
    zf = jnp.transpose(z, (0, 2, 3, 1)).reshape(-1, c)
    prob, idx3, loss = pl.pallas_call(
        _dist_body,
        grid=(_GRID,),
        in_specs=[
            pl.BlockSpec((_TILE, _DIM), lambda i: (i, 0)),
            pl.BlockSpec((_NUM_CODES, _DIM), lambda i: (0, 0)),
        ],
        out_specs=[
            pl.BlockSpec((_TILE, _NUM_CODES), lambda i: (i, 0)),
            pl.BlockSpec((1, 1, _TILE), lambda i: (i, 0, 0)),
            pl.BlockSpec(memory_space=pltpu.SMEM),
        ],
        out_shape=[
            jax.ShapeDtypeStruct((_ROWS, _NUM_CODES), jnp.float32),
            jax.ShapeDtypeStruct((_GRID, 1, _TILE), jnp.int32),
            jax.ShapeDtypeStruct((1, 1), jnp.float32),
        ],
        scratch_shapes=[pltpu.SMEM((1,), jnp.float32)],
        compiler_params=pltpu.CompilerParams(
            dimension_semantics=("arbitrary",)),
    )(zf, embedding)
    zq = _gather_fn()(embedding, idx3)
    zq_out = jnp.transpose(zq.reshape(b, h, w, c), (0, 3, 1, 2))
    return (zq_out, loss[0, 0], prob)


# speedup vs baseline: 1.3798x; 1.3798x over previous
import functools

import jax
import jax.numpy as jnp
from jax import lax
from jax.experimental import pallas as pl
from jax.experimental.pallas import tpu as pltpu
from jax.experimental.pallas import tpu_sc as plsc

_NUM_CODES = 8192
_DIM = 256
_ROWS = 8192
_TILE = 512
_GRID = _ROWS // _TILE
_HB = 8
_NC = 2
_NS = 16
_BPW = _ROWS // (_NC * _NS)
_CHUNK = 128

_LOG2E = 1.4426950408889634


def _dist_body(z_ref, e_ref, p_ref, idx_ref, loss_ref, acc_ref):
    z = z_ref[...]
    e = e_ref[...]
    z2 = jnp.sum(z * z, axis=1, keepdims=True)
    e2 = jnp.sum(e * e, axis=1)
    mm = lax.dot_general(z * -2.0, e, (((1,), (1,)), ((), ())),
                         preferred_element_type=jnp.float32)
    d = (z2 + e2[None, :]) + mm
    dmin = jnp.min(d, axis=1, keepdims=True)
    t = d - dmin
    ids8 = lax.broadcasted_iota(jnp.int32, (8, _NUM_CODES), 1).astype(
        jnp.float32)
    t3 = t.reshape(_TILE // 8, 8, _NUM_CODES)
    idxf = jnp.min(jnp.where(t3 == 0.0, ids8[None], jnp.float32(3e9)), axis=2)
    idx = idxf.reshape(_TILE).astype(jnp.int32)
    un = jnp.exp2(t * (-2.0 * _LOG2E))
    rcp = 1.0 / jnp.sum(un, axis=1, keepdims=True)
    p_ref[...] = un * rcp
    idx_ref[0, 0, :] = idx

    @pl.when(pl.program_id(0) == 0)
    def _():
        acc_ref[0] = 0.0

    acc_ref[0] += jnp.sum(dmin)

    @pl.when(pl.program_id(0) == _GRID - 1)
    def _():
        loss_ref[0, 0] = acc_ref[0] * (1.25 / (_ROWS * _DIM))


def _make_gather():
    mesh = plsc.VectorSubcoreMesh(core_axis_name="c", subcore_axis_name="s")

    @functools.partial(
        pl.kernel, mesh=mesh,
        out_type=jax.ShapeDtypeStruct((_ROWS, _DIM), jnp.float32),
        scratch_types=[
            pltpu.VMEM((_CHUNK,), jnp.int32),
            pltpu.VMEM((_CHUNK, _DIM), jnp.float32),
            pltpu.SemaphoreType.DMA,
        ],
    )
    def gather(emb_hbm, idx_hbm, out_hbm, idx_v, rows_v, sem):
        wid = lax.axis_index("s") * _NC + lax.axis_index("c")
        base = wid * _BPW
        for j in range(_BPW // _CHUNK):
            off = base + j * _CHUNK
            pltpu.sync_copy(
                idx_hbm.at[off // _TILE, 0, pl.ds(off % _TILE, _CHUNK)],
                idx_v)
            pltpu.async_copy(emb_hbm.at[idx_v], rows_v, sem).wait()
            pltpu.sync_copy(rows_v, out_hbm.at[pl.ds(off, _CHUNK)])

    return gather


@functools.cache
def _gather_fn():
    return _make_gather()


def kernel(z, embedding):
    b, c, h, w = z.shape
    zf = jnp.transpose(z, (0, 2, 3, 1)).reshape(-1, c)
    prob, idx3, loss = pl.pallas_call(
        _dist_body,
        grid=(_GRID,),
        in_specs=[
            pl.BlockSpec((_TILE, _DIM), lambda i: (i, 0)),
            pl.BlockSpec((_NUM_CODES, _DIM), lambda i: (0, 0)),
        ],
        out_specs=[
            pl.BlockSpec((_TILE, _NUM_CODES), lambda i: (i, 0)),
            pl.BlockSpec((1, 1, _TILE), lambda i: (i, 0, 0)),
            pl.BlockSpec(memory_space=pltpu.SMEM),
        ],
        out_shape=[
            jax.ShapeDtypeStruct((_ROWS, _NUM_CODES), jnp.float32),
            jax.ShapeDtypeStruct((_GRID, 1, _TILE), jnp.int32),
            jax.ShapeDtypeStruct((1, 1), jnp.float32),
        ],
        scratch_shapes=[pltpu.SMEM((1,), jnp.float32)],
        compiler_params=pltpu.CompilerParams(
            dimension_semantics=("arbitrary",)),
    )(zf, embedding)
    zq = _gather_fn()(embedding, idx3)
    zq_out = jnp.transpose(zq.reshape(b, h, w, c), (0, 3, 1, 2))
    return (zq_out, loss[0, 0], prob)
